# Initial kernel scaffold; baseline (speedup 1.0000x reference)
import jax
import jax.numpy as jnp
from jax.experimental import pallas as pl


def kernel(x, edge_index, W, b):
    raise NotImplementedError("write your pallas kernel here")


# speedup vs baseline: 15.7666x; 15.7666x over previous
import functools

import jax
import jax.numpy as jnp
from jax import lax
from jax.experimental import pallas as pl
from jax.experimental.pallas import tpu as pltpu
from jax.experimental.pallas import tpu_sc as plsc

NC = 2
NS = 16
L = 16
NW = NC * NS
K = 128


def _mesh():
    return plsc.VectorSubcoreMesh(core_axis_name="c", subcore_axis_name="s")


def _make_deg_kernel(NP, CW):
    EW = CW * K
    assert EW % L == 0

    @functools.partial(
        pl.kernel,
        out_type=jax.ShapeDtypeStruct((NW, NP), jnp.float32),
        mesh=_mesh(),
        scratch_types=[
            pltpu.VMEM((EW // L, L), jnp.int32),
            pltpu.VMEM((NP,), jnp.float32),
        ],
        compiler_params=pltpu.CompilerParams(needs_layout_passes=False),
    )
    def deg_kernel(row_hbm, out_hbm, idx_v, acc_v):
        c = lax.axis_index("c")
        s = lax.axis_index("s")
        wid = s * NC + c
        pltpu.sync_copy(row_hbm.at[wid], idx_v)
        zero = jnp.zeros((L,), jnp.float32)

        def zbody(j, carry):
            acc_v[pl.ds(j * L, L)] = zero
            return carry

        lax.fori_loop(0, NP // L, zbody, 0)
        ones = jnp.ones((L,), jnp.float32)

        def body(j, carry):
            idx = idx_v[j, :]
            plsc.addupdate_scatter(acc_v, [idx], ones)
            return carry

        lax.fori_loop(0, EW // L, body, 0)
        pltpu.sync_copy(acc_v, out_hbm.at[wid])

    return deg_kernel


def _make_edge_kernel(NP, CW):
    RS = NP // NS
    D = 128

    @functools.partial(
        pl.kernel,
        out_type=jax.ShapeDtypeStruct((NC * NP, D), jnp.float32),
        mesh=_mesh(),
        scratch_types=[
            pltpu.VMEM((CW, K), jnp.int32),
            pltpu.VMEM((CW, K), jnp.int32),
            pltpu.VMEM((K, D), jnp.float32),
            pltpu.VMEM_SHARED((NP, D), jnp.float32),
            pltpu.SemaphoreType.DMA,
        ],
    )
    def edge_kernel(g_hbm, ridx_hbm, cidx_hbm, out_hbm, ridx_v, cidx_v,
                    rows_v, acc_sh, sem):
        c = lax.axis_index("c")
        s = lax.axis_index("s")
        wid = s * NC + c
        pltpu.sync_copy(g_hbm.at[pl.ds(s * RS, RS)],
                        acc_sh.at[pl.ds(s * RS, RS)])
        pltpu.sync_copy(ridx_hbm.at[wid], ridx_v)
        pltpu.sync_copy(cidx_hbm.at[wid], cidx_v)
        plsc.subcore_barrier()

        def body(j, carry):
            pltpu.async_copy(g_hbm.at[ridx_v.at[j]], rows_v, sem).wait()
            pltpu.sync_copy(rows_v, acc_sh.at[cidx_v.at[j]], add=True)
            return carry

        lax.fori_loop(0, CW, body, 0)
        plsc.subcore_barrier()
        pltpu.sync_copy(acc_sh.at[pl.ds(s * RS, RS)],
                        out_hbm.at[pl.ds(c * NP + s * RS, RS)])

    return edge_kernel


def _pre_kernel(x_ref, w_ref, b_ref, degp_ref, g_ref, d_ref):
    h = lax.dot_general(x_ref[...], w_ref[...], (((1,), (1,)), ((), ())),
                        preferred_element_type=jnp.float32)
    h = h + b_ref[...]
    deg = jnp.sum(degp_ref[...], axis=0) + 1.0
    d = lax.rsqrt(deg)
    g_ref[...] = h * d[:, None]
    d_ref[...] = d[:, None]


def _post_kernel(p_ref, g_ref, d_ref, o_ref):
    acc = p_ref[0] + p_ref[1] - g_ref[...]
    o_ref[...] = acc * d_ref[...]


def kernel(x, edge_index, W, b):
    N, D_IN = x.shape
    D = W.shape[0]
    E = edge_index.shape[1]
    NP = 10240
    assert N < NP
    CW = -(-E // (NW * K))
    E_pad = NW * CW * K
    pad = E_pad - E

    row = edge_index[0]
    col = edge_index[1]
    row_d = jnp.concatenate([row, jnp.full((pad,), N, jnp.int32)])
    row_s = jnp.concatenate([row, jnp.zeros((pad,), jnp.int32)])
    col_s = jnp.concatenate([col, jnp.full((pad,), N, jnp.int32)])
    x_pad = jnp.pad(x, ((0, NP - N), (0, 0)))

    degp = _make_deg_kernel(NP, CW)(row_d.reshape(NW, CW * K // L, L))

    BR = 512
    grid = (NP // BR,)
    g, d = pl.pallas_call(
        _pre_kernel,
        grid=grid,
        in_specs=[
            pl.BlockSpec((BR, D_IN), lambda i: (i, 0)),
            pl.BlockSpec((D, D_IN), lambda i: (0, 0)),
            pl.BlockSpec((1, D), lambda i: (0, 0)),
            pl.BlockSpec((NW, BR), lambda i: (0, i)),
        ],
        out_specs=[
            pl.BlockSpec((BR, D), lambda i: (i, 0)),
            pl.BlockSpec((BR, 1), lambda i: (i, 0)),
        ],
        out_shape=[
            jax.ShapeDtypeStruct((NP, D), jnp.float32),
            jax.ShapeDtypeStruct((NP, 1), jnp.float32),
        ],
    )(x_pad, W, b.reshape(1, D), degp)

    p = _make_edge_kernel(NP, CW)(
        g, row_s.reshape(NW, CW, K), col_s.reshape(NW, CW, K))

    out = pl.pallas_call(
        _post_kernel,
        grid=grid,
        in_specs=[
            pl.BlockSpec((2, BR, D), lambda i: (0, i, 0)),
            pl.BlockSpec((BR, D), lambda i: (i, 0)),
            pl.BlockSpec((BR, 1), lambda i: (i, 0)),
        ],
        out_specs=pl.BlockSpec((BR, D), lambda i: (i, 0)),
        out_shape=jax.ShapeDtypeStruct((NP, D), jnp.float32),
    )(p.reshape(2, NP, D), g, d)

    return out[:N]


# speedup vs baseline: 19.8561x; 1.2594x over previous
import functools

import jax
import jax.numpy as jnp
from jax import lax
from jax.experimental import pallas as pl
from jax.experimental.pallas import tpu as pltpu
from jax.experimental.pallas import tpu_sc as plsc

NC = 2
NS = 16
L = 16
NW = NC * NS
K = 128


def _mesh():
    return plsc.VectorSubcoreMesh(core_axis_name="c", subcore_axis_name="s")


def _make_deg_kernel(NP, EW):
    assert EW % (L * 4) == 0

    @functools.partial(
        pl.kernel,
        out_type=jax.ShapeDtypeStruct((NW, NP), jnp.float32),
        mesh=_mesh(),
        scratch_types=[
            pltpu.VMEM((EW // L, L), jnp.int32),
            pltpu.VMEM((NP,), jnp.float32),
        ],
        compiler_params=pltpu.CompilerParams(needs_layout_passes=False),
    )
    def deg_kernel(row_hbm, out_hbm, idx_v, acc_v):
        c = lax.axis_index("c")
        s = lax.axis_index("s")
        wid = s * NC + c
        pltpu.sync_copy(row_hbm.at[wid], idx_v)
        zero = jnp.zeros((L,), jnp.float32)

        def zbody(j, carry):
            acc_v[pl.ds(j * L, L)] = zero
            return carry

        lax.fori_loop(0, NP // L, zbody, 0)
        ones = jnp.ones((L,), jnp.float32)

        def body(j, carry):
            for u in range(4):
                idx = idx_v[j * 4 + u, :]
                plsc.addupdate_scatter(acc_v, [idx], ones)
            return carry

        lax.fori_loop(0, EW // (L * 4), body, 0)
        pltpu.sync_copy(acc_v, out_hbm.at[wid])

    return deg_kernel


def _make_edge_kernel(NP, CW):
    RS = NP // NS
    D = 128
    assert CW % 2 == 1 and CW >= 3

    @functools.partial(
        pl.kernel,
        out_type=jax.ShapeDtypeStruct((NC * NP, D), jnp.float32),
        mesh=_mesh(),
        scratch_types=[
            pltpu.VMEM((CW, K), jnp.int32),
            pltpu.VMEM((2, K), jnp.int32),
            pltpu.VMEM((K, D), jnp.float32),
            pltpu.VMEM((K, D), jnp.float32),
            pltpu.VMEM_SHARED((NP, D), jnp.float32),
            pltpu.SemaphoreType.DMA,
            pltpu.SemaphoreType.DMA,
            pltpu.SemaphoreType.DMA,
            pltpu.SemaphoreType.DMA,
        ],
    )
    def edge_kernel(g_hbm, ridx_hbm, cidx_hbm, out_hbm, ridx_v, cring,
                    rows_a, rows_b, acc_sh, sem_a, sem_b, sem_c0, sem_c1):
        c = lax.axis_index("c")
        s = lax.axis_index("s")
        wid = s * NC + c
        cw = cidx_hbm.at[wid]
        pltpu.sync_copy(ridx_hbm.at[wid], ridx_v)
        pltpu.async_copy(g_hbm.at[ridx_v.at[0]], rows_a, sem_a)
        pltpu.async_copy(cw.at[0], cring.at[0], sem_c0)
        pltpu.async_copy(cw.at[1], cring.at[1], sem_c1)
        pltpu.sync_copy(g_hbm.at[pl.ds(s * RS, RS)],
                        acc_sh.at[pl.ds(s * RS, RS)])
        plsc.subcore_barrier()

        def body(i, carry):
            j0 = 2 * i
            pltpu.make_async_copy(g_hbm.at[ridx_v.at[j0]], rows_a,
                                  sem_a).wait()
            pltpu.async_copy(g_hbm.at[ridx_v.at[j0 + 1]], rows_b, sem_b)
            pltpu.make_async_copy(cw.at[j0], cring.at[0], sem_c0).wait()
            pltpu.sync_copy(rows_a, acc_sh.at[cring.at[0]], add=True)
            pltpu.async_copy(cw.at[j0 + 2], cring.at[0], sem_c0)
            pltpu.make_async_copy(g_hbm.at[ridx_v.at[j0 + 1]], rows_b,
                                  sem_b).wait()
            pltpu.async_copy(g_hbm.at[ridx_v.at[j0 + 2]], rows_a, sem_a)
            pltpu.make_async_copy(cw.at[j0 + 1], cring.at[1], sem_c1).wait()
            pltpu.sync_copy(rows_b, acc_sh.at[cring.at[1]], add=True)
            pltpu.async_copy(cw.at[j0 + 3], cring.at[1], sem_c1)
            return carry

        lax.fori_loop(0, CW // 2, body, 0)
        jl = CW - 1
        pltpu.make_async_copy(g_hbm.at[ridx_v.at[jl]], rows_a, sem_a).wait()
        pltpu.make_async_copy(cw.at[jl], cring.at[0], sem_c0).wait()
        pltpu.sync_copy(rows_a, acc_sh.at[cring.at[0]], add=True)
        pltpu.make_async_copy(cw.at[jl + 1], cring.at[1], sem_c1).wait()
        plsc.subcore_barrier()
        pltpu.sync_copy(acc_sh.at[pl.ds(s * RS, RS)],
                        out_hbm.at[pl.ds(c * NP + s * RS, RS)])

    return edge_kernel


def _pre_kernel(x_ref, w_ref, b_ref, degp_ref, g_ref, d_ref):
    h = lax.dot_general(x_ref[...], w_ref[...], (((1,), (1,)), ((), ())),
                        preferred_element_type=jnp.float32)
    h = h + b_ref[...]
    deg = jnp.sum(degp_ref[...], axis=0) + 1.0
    d = lax.rsqrt(deg)
    g_ref[...] = h * d[:, None]
    d_ref[...] = d[:, None]


def _post_kernel(p_ref, g_ref, d_ref, o_ref):
    acc = p_ref[0] + p_ref[1] - g_ref[...]
    o_ref[...] = acc * d_ref[...]


def kernel(x, edge_index, W, b):
    N, D_IN = x.shape
    D = W.shape[0]
    E = edge_index.shape[1]
    NP = 10240
    assert N < NP
    CW = -(-E // (NW * K))
    if CW % 2 == 0:
        CW += 1
    E_pad = NW * CW * K
    pad = E_pad - E
    EW_d = -(-E // (NW * L * 4)) * L * 4
    pad_d = NW * EW_d - E

    row = edge_index[0]
    col = edge_index[1]
    row_d = jnp.concatenate([row, jnp.full((pad_d,), N, jnp.int32)])
    row_s = jnp.concatenate([row, jnp.zeros((pad,), jnp.int32)])
    col_s = jnp.concatenate([col, jnp.full((pad,), N, jnp.int32)])
    x_pad = jnp.pad(x, ((0, NP - N), (0, 0)))

    degp = _make_deg_kernel(NP, EW_d)(row_d.reshape(NW, EW_d // L, L))

    BR = 512
    grid = (NP // BR,)
    g, d = pl.pallas_call(
        _pre_kernel,
        grid=grid,
        in_specs=[
            pl.BlockSpec((BR, D_IN), lambda i: (i, 0)),
            pl.BlockSpec((D, D_IN), lambda i: (0, 0)),
            pl.BlockSpec((1, D), lambda i: (0, 0)),
            pl.BlockSpec((NW, BR), lambda i: (0, i)),
        ],
        out_specs=[
            pl.BlockSpec((BR, D), lambda i: (i, 0)),
            pl.BlockSpec((BR, 1), lambda i: (i, 0)),
        ],
        out_shape=[
            jax.ShapeDtypeStruct((NP, D), jnp.float32),
            jax.ShapeDtypeStruct((NP, 1), jnp.float32),
        ],
    )(x_pad, W, b.reshape(1, D), degp)

    cidx3 = jnp.pad(col_s.reshape(NW, CW, K), ((0, 0), (0, 1), (0, 0)))
    p = _make_edge_kernel(NP, CW)(g, row_s.reshape(NW, CW, K), cidx3)

    out = pl.pallas_call(
        _post_kernel,
        grid=grid,
        in_specs=[
            pl.BlockSpec((2, BR, D), lambda i: (0, i, 0)),
            pl.BlockSpec((BR, D), lambda i: (i, 0)),
            pl.BlockSpec((BR, 1), lambda i: (i, 0)),
        ],
        out_specs=pl.BlockSpec((BR, D), lambda i: (i, 0)),
        out_shape=jax.ShapeDtypeStruct((NP, D), jnp.float32),
    )(p.reshape(2, NP, D), g, d)

    return out[:N]
